# Initial kernel scaffold; baseline (speedup 1.0000x reference)
import jax
import jax.numpy as jnp
from jax.experimental import pallas as pl


def kernel(x, mask):
    raise NotImplementedError("write your pallas kernel here")


# speedup vs baseline: 2.0545x; 2.0545x over previous
import jax
import jax.numpy as jnp
from jax.experimental import pallas as pl
from jax.experimental.pallas import tpu as pltpu

_BR = 512
_BC = 2048


def _body(x_ref, m_ref, o_ref, acc_ref):
    j = pl.program_id(1)

    @pl.when(j == 0)
    def _():
        acc_ref[...] = jnp.zeros_like(acc_ref)

    masked = x_ref[...] * m_ref[...].astype(jnp.float32)
    c = masked
    rows_b, cols_b = c.shape
    k = 1
    while k < cols_b:
        c = c + jnp.concatenate(
            [jnp.zeros((rows_b, k), jnp.float32), c[:, :-k]], axis=1
        )
        k *= 2
    out = c + acc_ref[...][:, :1]
    o_ref[...] = out
    acc_ref[...] = jnp.broadcast_to(out[:, -1:], acc_ref.shape)


def kernel(x, mask):
    rows, cols = x.shape
    grid = (rows // _BR, cols // _BC)
    return pl.pallas_call(
        _body,
        grid=grid,
        in_specs=[
            pl.BlockSpec((_BR, _BC), lambda i, j: (i, j)),
            pl.BlockSpec((_BR, _BC), lambda i, j: (i, j)),
        ],
        out_specs=pl.BlockSpec((_BR, _BC), lambda i, j: (i, j)),
        out_shape=jax.ShapeDtypeStruct((rows, cols), jnp.float32),
        scratch_shapes=[pltpu.VMEM((_BR, 128), jnp.float32)],
        compiler_params=pltpu.CompilerParams(
            dimension_semantics=("parallel", "arbitrary"),
        ),
    )(x, mask)


# speedup vs baseline: 3.6511x; 1.7771x over previous
import jax
import jax.numpy as jnp
from jax.experimental import pallas as pl
from jax.experimental.pallas import tpu as pltpu

_BR = 512
_BC = 2048
_G = 256


def _scan_small(a):
    w = a.shape[1]
    k = 1
    while k < w:
        a = a + jnp.concatenate(
            [jnp.zeros((a.shape[0], k), a.dtype), a[:, :-k]], axis=1
        )
        k *= 2
    return a


def _body(x_ref, m_ref, o_ref, acc_ref):
    j = pl.program_id(1)

    @pl.when(j == 0)
    def _():
        acc_ref[...] = jnp.zeros_like(acc_ref)

    masked = x_ref[...] * m_ref[...].astype(jnp.float32)
    ng = _BC // _G
    row = jax.lax.broadcasted_iota(jnp.int32, (_G, _G), 0)
    col = jax.lax.broadcasted_iota(jnp.int32, (_G, _G), 1)
    tri = (row <= col).astype(jnp.float32)

    local = [
        jnp.dot(masked[:, g * _G:(g + 1) * _G], tri,
                preferred_element_type=jnp.float32)
        for g in range(ng)
    ]
    gs = jnp.concatenate([l[:, _G - 1:_G] for l in local], axis=1)
    incl = _scan_small(gs)
    offs = incl - gs + acc_ref[...][:, :1]

    for g in range(ng):
        o_ref[:, g * _G:(g + 1) * _G] = local[g] + offs[:, g:g + 1]

    total = offs[:, ng - 1:ng] + gs[:, ng - 1:ng]
    acc_ref[...] = jnp.broadcast_to(total, acc_ref.shape)


def kernel(x, mask):
    rows, cols = x.shape
    grid = (rows // _BR, cols // _BC)
    return pl.pallas_call(
        _body,
        grid=grid,
        in_specs=[
            pl.BlockSpec((_BR, _BC), lambda i, j: (i, j)),
            pl.BlockSpec((_BR, _BC), lambda i, j: (i, j)),
        ],
        out_specs=pl.BlockSpec((_BR, _BC), lambda i, j: (i, j)),
        out_shape=jax.ShapeDtypeStruct((rows, cols), jnp.float32),
        scratch_shapes=[pltpu.VMEM((_BR, 128), jnp.float32)],
        compiler_params=pltpu.CompilerParams(
            dimension_semantics=("parallel", "arbitrary"),
        ),
    )(x, mask)
